# baseline (speedup 1.0000x reference)
import jax
import jax.numpy as jnp
from jax.experimental import pallas as pl
from jax.experimental.pallas import tpu as pltpu

_NUM_EMB = 512
_EMB_DIM = 64
_BETA = 0.25
_TB = 256
_CCHUNK = 256


def _fold_half(xm, tTg, base):
    def F(s):
        f = None
        for j in range(4):
            e = base + 8 * j + s
            u = jnp.abs(xm[:, e:e + 1] - tTg[e:e + 1, :])
            f = u if f is None else f + u
        return f

    t1 = F(0) + F(4)
    t2 = F(2) + F(6)
    t12 = t1 + t2
    t3 = F(1) + F(5)
    t4 = F(3) + F(7)
    return t12 + (t3 + t4)


def _block(s_ref, x_ref, t_ref, tT_ref, o_ref, loss_ref):
    i = pl.program_id(0)
    M = s_ref[0]
    rows = i * _TB + jax.lax.broadcasted_iota(jnp.int32, (_TB, 1), 0)

    @pl.when(i * _TB < M)
    def _compute():
        xm = jnp.where(rows >= M, 0.0, x_ref[...])

        best_d = None
        best_i = None
        iota_c = jax.lax.broadcasted_iota(jnp.int32, (_TB, _CCHUNK), 1)
        for g in range(_NUM_EMB // _CCHUNK):
            tTg = tT_ref[:, g * _CCHUNK:(g + 1) * _CCHUNK]
            dg = _fold_half(xm, tTg, 0) + _fold_half(xm, tTg, 32)
            mg = jnp.min(dg, axis=1, keepdims=True)
            ig = jnp.min(jnp.where(dg == mg, iota_c + g * _CCHUNK, _NUM_EMB),
                         axis=1, keepdims=True)
            if g == 0:
                best_d, best_i = mg, ig
            else:
                take = mg < best_d
                best_i = jnp.where(take, ig, best_i)
                best_d = jnp.where(take, mg, best_d)

        iota = jax.lax.broadcasted_iota(jnp.int32, (_TB, _NUM_EMB), 1)
        onehot = (iota == best_i).astype(jnp.float32)
        zq = jax.lax.dot_general(
            onehot, t_ref[...], (((1,), (0,)), ((), ())),
            precision=jax.lax.Precision.HIGHEST,
            preferred_element_type=jnp.float32)
        zq = jnp.where(rows >= M, 0.0, zq)

        o_ref[...] = xm + (zq - xm)
        diff = xm - zq
        part = jnp.sum(diff * diff).reshape(1, 1)

        @pl.when(i == 0)
        def _init():
            loss_ref[...] = part

        @pl.when(i != 0)
        def _acc():
            loss_ref[...] += part

    @pl.when(i * _TB >= M)
    def _skip():
        o_ref[...] = jnp.zeros((_TB, _EMB_DIM), jnp.float32)

        @pl.when(i == 0)
        def _empty():
            loss_ref[...] = jnp.zeros((1, 1), jnp.float32)


def kernel(x, mask, table):
    B, T, E = x.shape
    N = B * T
    xf = x.reshape(N, E)
    tT = jnp.swapaxes(table, 0, 1)

    mi = mask.reshape(N).astype(jnp.int32)
    keep = 1 - mi
    ck = jnp.cumsum(keep)
    M = ck[N - 1:]
    ar = jnp.arange(N, dtype=jnp.int32)
    dest = jnp.where(keep == 1, ck - 1, M[0] + ar - ck)
    order = jnp.argsort(mi, stable=True).astype(jnp.int32)
    xc = jnp.take(xf, order, axis=0)

    zc, losssum = pl.pallas_call(
        _block,
        grid_spec=pltpu.PrefetchScalarGridSpec(
            num_scalar_prefetch=1,
            grid=(N // _TB,),
            in_specs=[
                pl.BlockSpec((_TB, E), lambda i, s: (i, 0)),
                pl.BlockSpec((_NUM_EMB, E), lambda i, s: (0, 0)),
                pl.BlockSpec((E, _NUM_EMB), lambda i, s: (0, 0)),
            ],
            out_specs=[
                pl.BlockSpec((_TB, E), lambda i, s: (i, 0)),
                pl.BlockSpec((1, 1), lambda i, s: (0, 0)),
            ],
        ),
        out_shape=[
            jax.ShapeDtypeStruct((N, E), jnp.float32),
            jax.ShapeDtypeStruct((1, 1), jnp.float32),
        ],
    )(M, xc, table, tT)

    emb = losssum[0, 0] / (N * E)
    return (jnp.take(zc, dest, axis=0).reshape(B, T, E), emb, _BETA * emb)


# speedup vs baseline: 1.0877x; 1.0877x over previous
import jax
import jax.numpy as jnp
from jax.experimental import pallas as pl
from jax.experimental.pallas import tpu as pltpu

_NUM_EMB = 512
_EMB_DIM = 64
_BETA = 0.25
_TB = 512
_CCHUNK = 256


def _fold_half(xm, tTg, base):
    def F(s):
        f = None
        for j in range(4):
            e = base + 8 * j + s
            u = jnp.abs(xm[:, e:e + 1] - tTg[e:e + 1, :])
            f = u if f is None else f + u
        return f

    t1 = F(0) + F(4)
    t2 = F(2) + F(6)
    t12 = t1 + t2
    t3 = F(1) + F(5)
    t4 = F(3) + F(7)
    return t12 + (t3 + t4)


def _block(s_ref, x_ref, t_ref, tT_ref, o_ref, loss_ref):
    i = pl.program_id(0)
    M = s_ref[0]
    rows = i * _TB + jax.lax.broadcasted_iota(jnp.int32, (_TB, 1), 0)

    @pl.when(i * _TB < M)
    def _compute():
        xm = jnp.where(rows >= M, 0.0, x_ref[...])

        best_d = None
        best_i = None
        iota_c = jax.lax.broadcasted_iota(jnp.int32, (_TB, _CCHUNK), 1)
        for g in range(_NUM_EMB // _CCHUNK):
            tTg = tT_ref[:, g * _CCHUNK:(g + 1) * _CCHUNK]
            dg = _fold_half(xm, tTg, 0) + _fold_half(xm, tTg, 32)
            mg = jnp.min(dg, axis=1, keepdims=True)
            ig = jnp.min(jnp.where(dg == mg, iota_c + g * _CCHUNK, _NUM_EMB),
                         axis=1, keepdims=True)
            if g == 0:
                best_d, best_i = mg, ig
            else:
                take = mg < best_d
                best_i = jnp.where(take, ig, best_i)
                best_d = jnp.where(take, mg, best_d)

        iota = jax.lax.broadcasted_iota(jnp.int32, (_TB, _NUM_EMB), 1)
        onehot = (iota == best_i).astype(jnp.float32)
        zq = jax.lax.dot_general(
            onehot, t_ref[...], (((1,), (0,)), ((), ())),
            precision=jax.lax.Precision.HIGHEST,
            preferred_element_type=jnp.float32)
        zq = jnp.where(rows >= M, 0.0, zq)

        o_ref[...] = xm + (zq - xm)
        diff = xm - zq
        part = jnp.sum(diff * diff).reshape(1, 1)

        @pl.when(i == 0)
        def _init():
            loss_ref[...] = part

        @pl.when(i != 0)
        def _acc():
            loss_ref[...] += part

    @pl.when(i * _TB >= M)
    def _skip():
        o_ref[...] = jnp.zeros((_TB, _EMB_DIM), jnp.float32)

        @pl.when(i == 0)
        def _empty():
            loss_ref[...] = jnp.zeros((1, 1), jnp.float32)


def kernel(x, mask, table):
    B, T, E = x.shape
    N = B * T
    xf = x.reshape(N, E)
    tT = jnp.swapaxes(table, 0, 1)

    mi = mask.reshape(N).astype(jnp.int32)
    keep = 1 - mi
    ck = jnp.cumsum(keep)
    M = ck[N - 1:]
    ar = jnp.arange(N, dtype=jnp.int32)
    dest = jnp.where(keep == 1, ck - 1, M[0] + ar - ck)
    xc = jnp.zeros((N, E), jnp.float32).at[dest].set(xf)

    zc, losssum = pl.pallas_call(
        _block,
        grid_spec=pltpu.PrefetchScalarGridSpec(
            num_scalar_prefetch=1,
            grid=(N // _TB,),
            in_specs=[
                pl.BlockSpec((_TB, E), lambda i, s: (i, 0)),
                pl.BlockSpec((_NUM_EMB, E), lambda i, s: (0, 0)),
                pl.BlockSpec((E, _NUM_EMB), lambda i, s: (0, 0)),
            ],
            out_specs=[
                pl.BlockSpec((_TB, E), lambda i, s: (i, 0)),
                pl.BlockSpec((1, 1), lambda i, s: (0, 0)),
            ],
        ),
        out_shape=[
            jax.ShapeDtypeStruct((N, E), jnp.float32),
            jax.ShapeDtypeStruct((1, 1), jnp.float32),
        ],
    )(M, xc, table, tT)

    emb = losssum[0, 0] / (N * E)
    return (jnp.take(zc, dest, axis=0).reshape(B, T, E), emb, _BETA * emb)
